# Initial kernel scaffold; baseline (speedup 1.0000x reference)
import functools

import jax
import jax.numpy as jnp
from jax import lax
from jax.experimental import pallas as pl
from jax.experimental.pallas import tpu as pltpu
from jax.experimental.pallas import tpu_sc as plsc

N = 10000
E = 320000
D = 128

NC = 2
NS = 16
NW = NC * NS
EPW = E // NW
K = 80
CH = EPW // K
TR = N // NS


def _make_agg_kernel(compute_deg: bool):
    mesh = plsc.VectorSubcoreMesh(core_axis_name="c", subcore_axis_name="s")
    out_type = [jax.ShapeDtypeStruct((NC, N, D), jnp.float32)]
    if compute_deg:
        out_type.append(jax.ShapeDtypeStruct((NC, N, 1), jnp.float32))
    scratch = [
        pltpu.VMEM((CH, K), jnp.int32),
        pltpu.VMEM((CH, K), jnp.int32),
        pltpu.VMEM((K, D), jnp.float32),
        pltpu.VMEM((K, 1), jnp.float32),
        pltpu.VMEM_SHARED((N, D), jnp.float32),
        pltpu.VMEM_SHARED((N, 1), jnp.float32),
        pltpu.SemaphoreType.DMA,
    ]

    def body(h_hbm, srcr_hbm, dstr_hbm, zrows_hbm, zdeg_hbm, ones_hbm,
             agg_out, deg_out, src_v, dst_v, rows_v, ones_v, agg_sh,
             deg_sh, sem):
        c = lax.axis_index("c")
        s = lax.axis_index("s")
        wid = s * NC + c
        row0 = s * TR

        pltpu.sync_copy(zrows_hbm.at[pl.ds(row0, TR)],
                        agg_sh.at[pl.ds(row0, TR)])
        if compute_deg:
            pltpu.sync_copy(zdeg_hbm.at[pl.ds(row0, TR)],
                            deg_sh.at[pl.ds(row0, TR)])
            pltpu.sync_copy(ones_hbm, ones_v)
        pltpu.sync_copy(srcr_hbm.at[wid], src_v)
        pltpu.sync_copy(dstr_hbm.at[wid], dst_v)
        plsc.subcore_barrier()

        def step(g, carry):
            pltpu.async_copy(h_hbm.at[src_v.at[g]], rows_v, sem).wait()
            pltpu.sync_copy(rows_v, agg_sh.at[dst_v.at[g]], add=True)
            if compute_deg:
                pltpu.sync_copy(ones_v, deg_sh.at[dst_v.at[g]], add=True)
            return carry

        lax.fori_loop(0, CH, step, 0)
        plsc.subcore_barrier()

        pltpu.sync_copy(agg_sh.at[pl.ds(row0, TR)],
                        agg_out.at[c, pl.ds(row0, TR)])
        if compute_deg:
            pltpu.sync_copy(deg_sh.at[pl.ds(row0, TR)],
                            deg_out.at[c, pl.ds(row0, TR)])

    if compute_deg:
        fn = body
    else:
        def fn(h_hbm, srcr_hbm, dstr_hbm, zrows_hbm, agg_out, src_v, dst_v,
               rows_v, ones_v, agg_sh, deg_sh, sem):
            body(h_hbm, srcr_hbm, dstr_hbm, zrows_hbm, None, None, agg_out,
                 None, src_v, dst_v, rows_v, ones_v, agg_sh, deg_sh, sem)

    return pl.kernel(fn, out_type=tuple(out_type) if compute_deg else out_type[0],
                     mesh=mesh, scratch_types=scratch)


_agg_with_deg = _make_agg_kernel(True)
_agg_no_deg = _make_agg_kernel(False)


R = 1000


def _tc_layer0_body(feat_ref, agg_ref, deg_ref, ws_ref, wn_ref, b_ref,
                    g_ref, beta_ref, out_ref):
    aggp = agg_ref[...]
    agg = aggp[0] + aggp[1]
    degp = deg_ref[...]
    deg = degp[0] + degp[1]
    inv = 1.0 / jnp.maximum(deg, 1.0)
    mean = agg * inv
    h = (jnp.dot(feat_ref[...], ws_ref[...], preferred_element_type=jnp.float32)
         + jnp.dot(mean, wn_ref[...], preferred_element_type=jnp.float32)
         + b_ref[...])
    mu = jnp.mean(h, axis=-1, keepdims=True)
    var = jnp.mean((h - mu) ** 2, axis=-1, keepdims=True)
    hn = (h - mu) * lax.rsqrt(var + 1e-5) * g_ref[...] + beta_ref[...]
    out_ref[...] = jnp.maximum(hn, 0.0)


def _tc_layer1_body(h_ref, agg_ref, deg_ref, ws_ref, wn_ref, b_ref, out_ref):
    aggp = agg_ref[...]
    agg = aggp[0] + aggp[1]
    degp = deg_ref[...]
    deg = degp[0] + degp[1]
    inv = 1.0 / jnp.maximum(deg, 1.0)
    mean = agg * inv
    out_ref[...] = (
        jnp.dot(h_ref[...], ws_ref[...], preferred_element_type=jnp.float32)
        + jnp.dot(mean, wn_ref[...], preferred_element_type=jnp.float32)
        + b_ref[...])


def _full(shape):
    return pl.BlockSpec(shape, lambda i: (0,) * len(shape))


_row_spec = pl.BlockSpec((R, D), lambda i: (i, 0))
_agg_spec = pl.BlockSpec((NC, R, D), lambda i: (0, i, 0))
_deg_spec = pl.BlockSpec((NC, R, 1), lambda i: (0, i, 0))

_tc_layer0 = pl.pallas_call(
    _tc_layer0_body,
    grid=(N // R,),
    in_specs=[_row_spec, _agg_spec, _deg_spec, _full((D, D)), _full((D, D)),
              _full((1, D)), _full((1, D)), _full((1, D))],
    out_specs=_row_spec,
    out_shape=jax.ShapeDtypeStruct((N, D), jnp.float32),
)

_tc_layer1 = pl.pallas_call(
    _tc_layer1_body,
    grid=(N // R,),
    in_specs=[_row_spec, _agg_spec, _deg_spec, _full((D, D)), _full((D, D)),
              _full((1, D))],
    out_specs=_row_spec,
    out_shape=jax.ShapeDtypeStruct((N, D), jnp.float32),
)


def kernel(feat, edge_index, W0_self, W0_neigh, b0, ln_g, ln_b,
           W1_self, W1_neigh, b1):
    src = edge_index[0].astype(jnp.int32).reshape(NW, CH, K)
    dst = edge_index[1].astype(jnp.int32).reshape(NW, CH, K)
    zrows = jnp.zeros((N, D), jnp.float32)
    zdeg = jnp.zeros((N, 1), jnp.float32)
    ones = jnp.ones((K, 1), jnp.float32)

    agg0, deg = _agg_with_deg(feat, src, dst, zrows, zdeg, ones)
    h1 = _tc_layer0(feat, agg0, deg, W0_self, W0_neigh,
                    b0.reshape(1, D), ln_g.reshape(1, D), ln_b.reshape(1, D))
    agg1 = _agg_no_deg(h1, src, dst, zrows)
    out = _tc_layer1(h1, agg1, deg, W1_self, W1_neigh, b1.reshape(1, D))
    return out


# speedup vs baseline: 4.7822x; 4.7822x over previous
import functools

import jax
import jax.numpy as jnp
from jax import lax
from jax.experimental import pallas as pl
from jax.experimental.pallas import tpu as pltpu
from jax.experimental.pallas import tpu_sc as plsc

N = 10000
E = 320000
D = 128

NC = 2
NS = 16
NW = NC * NS
EPW = E // NW
K = 80
CH = EPW // K
TRB = 624
LAST0 = TRB * (NS - 1)
LASTR = N - LAST0


def _make_agg_kernel():
    mesh = plsc.VectorSubcoreMesh(core_axis_name="c", subcore_axis_name="s")
    scratch = [
        pltpu.VMEM((K,), jnp.int32),
        pltpu.VMEM((K,), jnp.int32),
        pltpu.VMEM((K, D), jnp.float32),
        pltpu.VMEM_SHARED((N, D), jnp.float32),
        pltpu.SemaphoreType.DMA,
    ]

    def body(h_hbm, srcr_hbm, dstr_hbm, zrows_hbm, agg_out,
             src_v, dst_v, rows_v, agg_sh, sem):
        c = lax.axis_index("c")
        s = lax.axis_index("s")
        wid = s * NC + c
        row0 = pl.multiple_of(s * TRB, 8)

        @pl.when(s < NS - 1)
        def _():
            pltpu.sync_copy(zrows_hbm.at[pl.ds(row0, TRB)],
                            agg_sh.at[pl.ds(row0, TRB)])

        @pl.when(s == NS - 1)
        def _():
            pltpu.sync_copy(zrows_hbm.at[pl.ds(LAST0, LASTR)],
                            agg_sh.at[pl.ds(LAST0, LASTR)])

        plsc.subcore_barrier()

        base = wid * EPW

        def step(g, carry):
            off = pl.multiple_of(base + g * K, 8)
            pltpu.sync_copy(srcr_hbm.at[pl.ds(off, K)], src_v)
            pltpu.sync_copy(dstr_hbm.at[pl.ds(off, K)], dst_v)
            pltpu.async_copy(h_hbm.at[src_v], rows_v, sem).wait()
            pltpu.sync_copy(rows_v, agg_sh.at[dst_v], add=True)
            return carry

        lax.fori_loop(0, CH, step, 0)
        plsc.subcore_barrier()

        @pl.when(s < NS - 1)
        def _():
            pltpu.sync_copy(agg_sh.at[pl.ds(row0, TRB)],
                            agg_out.at[c, pl.ds(row0, TRB)])

        @pl.when(s == NS - 1)
        def _():
            pltpu.sync_copy(agg_sh.at[pl.ds(LAST0, LASTR)],
                            agg_out.at[c, pl.ds(LAST0, LASTR)])

    return pl.kernel(body,
                     out_type=jax.ShapeDtypeStruct((NC, N, D), jnp.float32),
                     mesh=mesh, scratch_types=scratch)


def _make_deg_kernel():
    mesh = plsc.VectorSubcoreMesh(core_axis_name="c", subcore_axis_name="s")
    scratch = [
        pltpu.VMEM((K,), jnp.int32),
        pltpu.VMEM((K, D), jnp.float32),
        pltpu.VMEM_SHARED((N, D), jnp.float32),
    ]

    def body(dstr_hbm, zrows_hbm, ones_hbm, deg_out, dst_v, ones_v, deg_sh):
        c = lax.axis_index("c")
        s = lax.axis_index("s")
        wid = s * NC + c
        row0 = pl.multiple_of(s * TRB, 8)

        @pl.when(s < NS - 1)
        def _():
            pltpu.sync_copy(zrows_hbm.at[pl.ds(row0, TRB)],
                            deg_sh.at[pl.ds(row0, TRB)])

        @pl.when(s == NS - 1)
        def _():
            pltpu.sync_copy(zrows_hbm.at[pl.ds(LAST0, LASTR)],
                            deg_sh.at[pl.ds(LAST0, LASTR)])

        pltpu.sync_copy(ones_hbm, ones_v)
        plsc.subcore_barrier()

        base = wid * EPW

        def step(g, carry):
            off = pl.multiple_of(base + g * K, 8)
            pltpu.sync_copy(dstr_hbm.at[pl.ds(off, K)], dst_v)
            pltpu.sync_copy(ones_v, deg_sh.at[dst_v], add=True)
            return carry

        lax.fori_loop(0, CH, step, 0)
        plsc.subcore_barrier()

        @pl.when(s < NS - 1)
        def _():
            pltpu.sync_copy(deg_sh.at[pl.ds(row0, TRB)],
                            deg_out.at[c, pl.ds(row0, TRB)])

        @pl.when(s == NS - 1)
        def _():
            pltpu.sync_copy(deg_sh.at[pl.ds(LAST0, LASTR)],
                            deg_out.at[c, pl.ds(LAST0, LASTR)])

    return pl.kernel(body,
                     out_type=jax.ShapeDtypeStruct((NC, N, D), jnp.float32),
                     mesh=mesh, scratch_types=scratch)


_agg = _make_agg_kernel()
_deg = _make_deg_kernel()


R = 1000


def _tc_layer0_body(feat_ref, agg_ref, deg_ref, ws_ref, wn_ref, b_ref,
                    g_ref, beta_ref, out_ref):
    aggp = agg_ref[...]
    agg = aggp[0] + aggp[1]
    degp = deg_ref[...]
    deg = (degp[0] + degp[1])[:, 0:1]
    inv = 1.0 / jnp.maximum(deg, 1.0)
    mean = agg * inv
    h = (jnp.dot(feat_ref[...], ws_ref[...], preferred_element_type=jnp.float32)
         + jnp.dot(mean, wn_ref[...], preferred_element_type=jnp.float32)
         + b_ref[...])
    mu = jnp.mean(h, axis=-1, keepdims=True)
    var = jnp.mean((h - mu) ** 2, axis=-1, keepdims=True)
    hn = (h - mu) * lax.rsqrt(var + 1e-5) * g_ref[...] + beta_ref[...]
    out_ref[...] = jnp.maximum(hn, 0.0)


def _tc_layer1_body(h_ref, agg_ref, deg_ref, ws_ref, wn_ref, b_ref, out_ref):
    aggp = agg_ref[...]
    agg = aggp[0] + aggp[1]
    degp = deg_ref[...]
    deg = (degp[0] + degp[1])[:, 0:1]
    inv = 1.0 / jnp.maximum(deg, 1.0)
    mean = agg * inv
    out_ref[...] = (
        jnp.dot(h_ref[...], ws_ref[...], preferred_element_type=jnp.float32)
        + jnp.dot(mean, wn_ref[...], preferred_element_type=jnp.float32)
        + b_ref[...])


def _full(shape):
    return pl.BlockSpec(shape, lambda i: (0,) * len(shape))


_row_spec = pl.BlockSpec((R, D), lambda i: (i, 0))
_agg_spec = pl.BlockSpec((NC, R, D), lambda i: (0, i, 0))
_deg_spec = pl.BlockSpec((NC, R, D), lambda i: (0, i, 0))

_tc_layer0 = pl.pallas_call(
    _tc_layer0_body,
    grid=(N // R,),
    in_specs=[_row_spec, _agg_spec, _deg_spec, _full((D, D)), _full((D, D)),
              _full((1, D)), _full((1, D)), _full((1, D))],
    out_specs=_row_spec,
    out_shape=jax.ShapeDtypeStruct((N, D), jnp.float32),
)

_tc_layer1 = pl.pallas_call(
    _tc_layer1_body,
    grid=(N // R,),
    in_specs=[_row_spec, _agg_spec, _deg_spec, _full((D, D)), _full((D, D)),
              _full((1, D))],
    out_specs=_row_spec,
    out_shape=jax.ShapeDtypeStruct((N, D), jnp.float32),
)


def kernel(feat, edge_index, W0_self, W0_neigh, b0, ln_g, ln_b,
           W1_self, W1_neigh, b1):
    src = edge_index[0].astype(jnp.int32)
    dst = edge_index[1].astype(jnp.int32)
    zrows = jnp.zeros((N, D), jnp.float32)
    ones = jnp.ones((K, D), jnp.float32)

    deg = _deg(dst, zrows, ones)
    agg0 = _agg(feat, src, dst, zrows)
    h1 = _tc_layer0(feat, agg0, deg, W0_self, W0_neigh,
                    b0.reshape(1, D), ln_g.reshape(1, D), ln_b.reshape(1, D))
    agg1 = _agg(h1, src, dst, zrows)
    out = _tc_layer1(h1, agg1, deg, W1_self, W1_neigh, b1.reshape(1, D))
    return out


# speedup vs baseline: 5.9009x; 1.2339x over previous
import functools

import jax
import jax.numpy as jnp
from jax import lax
from jax.experimental import pallas as pl
from jax.experimental.pallas import tpu as pltpu
from jax.experimental.pallas import tpu_sc as plsc

N = 10000
E = 320000
D = 128

NC = 2
NS = 16
NW = NC * NS
EPW = E // NW
K = 80
CH = EPW // K
TRB = 624
LAST0 = TRB * (NS - 1)
LASTR = N - LAST0


def _make_agg_kernel():
    mesh = plsc.VectorSubcoreMesh(core_axis_name="c", subcore_axis_name="s")
    scratch = [
        pltpu.VMEM((2, K), jnp.int32),
        pltpu.VMEM((2, K), jnp.int32),
        pltpu.VMEM((2, K, D), jnp.float32),
        pltpu.VMEM_SHARED((N, D), jnp.float32),
        pltpu.SemaphoreType.DMA,
        pltpu.SemaphoreType.DMA,
        pltpu.SemaphoreType.DMA,
        pltpu.SemaphoreType.DMA,
    ]

    def body(h_hbm, srcr_hbm, dstr_hbm, zrows_hbm, agg_out,
             src_v, dst_v, rows_v, agg_sh, sg0, sg1, ss0, ss1):
        c = lax.axis_index("c")
        s = lax.axis_index("s")
        wid = s * NC + c
        row0 = pl.multiple_of(s * TRB, 8)

        @pl.when(s < NS - 1)
        def _():
            pltpu.sync_copy(zrows_hbm.at[pl.ds(row0, TRB)],
                            agg_sh.at[pl.ds(row0, TRB)])

        @pl.when(s == NS - 1)
        def _():
            pltpu.sync_copy(zrows_hbm.at[pl.ds(LAST0, LASTR)],
                            agg_sh.at[pl.ds(LAST0, LASTR)])

        plsc.subcore_barrier()

        base = wid * EPW
        sg = (sg0, sg1)
        ss = (ss0, ss1)

        def load_idx(b, chunk):
            off = pl.multiple_of(base + chunk * K, 8)
            pltpu.sync_copy(srcr_hbm.at[pl.ds(off, K)], src_v.at[b])
            pltpu.sync_copy(dstr_hbm.at[pl.ds(off, K)], dst_v.at[b])

        def fire_gather(b):
            pltpu.async_copy(h_hbm.at[src_v.at[b]], rows_v.at[b], sg[b])

        def wait_gather(b):
            pltpu.make_async_copy(h_hbm.at[src_v.at[b]], rows_v.at[b],
                                  sg[b]).wait()

        def fire_scatter(b):
            pltpu.async_copy(rows_v.at[b], agg_sh.at[dst_v.at[b]], ss[b],
                             add=True)

        def wait_scatter(b):
            pltpu.make_async_copy(rows_v.at[b], agg_sh.at[dst_v.at[b]],
                                  ss[b]).wait()

        load_idx(0, 0)
        fire_gather(0)

        def step(g2, carry):
            g0 = 2 * g2
            wait_gather(0)
            fire_scatter(0)

            @pl.when(g2 > 0)
            def _():
                wait_scatter(1)
            load_idx(1, g0 + 1)
            fire_gather(1)
            wait_gather(1)
            fire_scatter(1)
            wait_scatter(0)
            load_idx(0, g0 + 2)
            fire_gather(0)
            return carry

        lax.fori_loop(0, (CH - 1) // 2, step, 0)
        wait_gather(0)
        fire_scatter(0)
        wait_scatter(1)
        wait_scatter(0)
        plsc.subcore_barrier()

        @pl.when(s < NS - 1)
        def _():
            pltpu.sync_copy(agg_sh.at[pl.ds(row0, TRB)],
                            agg_out.at[c, pl.ds(row0, TRB)])

        @pl.when(s == NS - 1)
        def _():
            pltpu.sync_copy(agg_sh.at[pl.ds(LAST0, LASTR)],
                            agg_out.at[c, pl.ds(LAST0, LASTR)])

    return pl.kernel(body,
                     out_type=jax.ShapeDtypeStruct((NC, N, D), jnp.float32),
                     mesh=mesh, scratch_types=scratch)


def _make_deg_kernel():
    mesh = plsc.VectorSubcoreMesh(core_axis_name="c", subcore_axis_name="s")
    scratch = [
        pltpu.VMEM((2, K), jnp.int32),
        pltpu.VMEM((K, D), jnp.float32),
        pltpu.VMEM_SHARED((N, D), jnp.float32),
        pltpu.SemaphoreType.DMA,
        pltpu.SemaphoreType.DMA,
    ]

    def body(dstr_hbm, zrows_hbm, ones_hbm, deg_out, dst_v, ones_v, deg_sh,
             ss0, ss1):
        c = lax.axis_index("c")
        s = lax.axis_index("s")
        wid = s * NC + c
        row0 = pl.multiple_of(s * TRB, 8)

        @pl.when(s < NS - 1)
        def _():
            pltpu.sync_copy(zrows_hbm.at[pl.ds(row0, TRB)],
                            deg_sh.at[pl.ds(row0, TRB)])

        @pl.when(s == NS - 1)
        def _():
            pltpu.sync_copy(zrows_hbm.at[pl.ds(LAST0, LASTR)],
                            deg_sh.at[pl.ds(LAST0, LASTR)])

        pltpu.sync_copy(ones_hbm, ones_v)
        plsc.subcore_barrier()

        base = wid * EPW
        ss = (ss0, ss1)

        def load_idx(b, chunk):
            off = pl.multiple_of(base + chunk * K, 8)
            pltpu.sync_copy(dstr_hbm.at[pl.ds(off, K)], dst_v.at[b])

        def fire_scatter(b):
            pltpu.async_copy(ones_v, deg_sh.at[dst_v.at[b]], ss[b], add=True)

        def wait_scatter(b):
            pltpu.make_async_copy(ones_v, deg_sh.at[dst_v.at[b]], ss[b]).wait()

        load_idx(0, 0)

        def step(g2, carry):
            g0 = 2 * g2
            fire_scatter(0)

            @pl.when(g2 > 0)
            def _():
                wait_scatter(1)
            load_idx(1, g0 + 1)
            fire_scatter(1)
            wait_scatter(0)
            load_idx(0, g0 + 2)
            return carry

        lax.fori_loop(0, (CH - 1) // 2, step, 0)
        fire_scatter(0)
        wait_scatter(1)
        wait_scatter(0)
        plsc.subcore_barrier()

        @pl.when(s < NS - 1)
        def _():
            pltpu.sync_copy(deg_sh.at[pl.ds(row0, TRB)],
                            deg_out.at[c, pl.ds(row0, TRB)])

        @pl.when(s == NS - 1)
        def _():
            pltpu.sync_copy(deg_sh.at[pl.ds(LAST0, LASTR)],
                            deg_out.at[c, pl.ds(LAST0, LASTR)])

    return pl.kernel(body,
                     out_type=jax.ShapeDtypeStruct((NC, N, D), jnp.float32),
                     mesh=mesh, scratch_types=scratch)


_agg = _make_agg_kernel()
_deg = _make_deg_kernel()


R = 1000


def _tc_layer0_body(feat_ref, agg_ref, deg_ref, ws_ref, wn_ref, b_ref,
                    g_ref, beta_ref, out_ref):
    aggp = agg_ref[...]
    agg = aggp[0] + aggp[1]
    degp = deg_ref[...]
    deg = (degp[0] + degp[1])[:, 0:1]
    inv = 1.0 / jnp.maximum(deg, 1.0)
    mean = agg * inv
    h = (jnp.dot(feat_ref[...], ws_ref[...], preferred_element_type=jnp.float32)
         + jnp.dot(mean, wn_ref[...], preferred_element_type=jnp.float32)
         + b_ref[...])
    mu = jnp.mean(h, axis=-1, keepdims=True)
    var = jnp.mean((h - mu) ** 2, axis=-1, keepdims=True)
    hn = (h - mu) * lax.rsqrt(var + 1e-5) * g_ref[...] + beta_ref[...]
    out_ref[...] = jnp.maximum(hn, 0.0)


def _tc_layer1_body(h_ref, agg_ref, deg_ref, ws_ref, wn_ref, b_ref, out_ref):
    aggp = agg_ref[...]
    agg = aggp[0] + aggp[1]
    degp = deg_ref[...]
    deg = (degp[0] + degp[1])[:, 0:1]
    inv = 1.0 / jnp.maximum(deg, 1.0)
    mean = agg * inv
    out_ref[...] = (
        jnp.dot(h_ref[...], ws_ref[...], preferred_element_type=jnp.float32)
        + jnp.dot(mean, wn_ref[...], preferred_element_type=jnp.float32)
        + b_ref[...])


def _full(shape):
    return pl.BlockSpec(shape, lambda i: (0,) * len(shape))


_row_spec = pl.BlockSpec((R, D), lambda i: (i, 0))
_agg_spec = pl.BlockSpec((NC, R, D), lambda i: (0, i, 0))
_deg_spec = pl.BlockSpec((NC, R, D), lambda i: (0, i, 0))

_tc_layer0 = pl.pallas_call(
    _tc_layer0_body,
    grid=(N // R,),
    in_specs=[_row_spec, _agg_spec, _deg_spec, _full((D, D)), _full((D, D)),
              _full((1, D)), _full((1, D)), _full((1, D))],
    out_specs=_row_spec,
    out_shape=jax.ShapeDtypeStruct((N, D), jnp.float32),
)

_tc_layer1 = pl.pallas_call(
    _tc_layer1_body,
    grid=(N // R,),
    in_specs=[_row_spec, _agg_spec, _deg_spec, _full((D, D)), _full((D, D)),
              _full((1, D))],
    out_specs=_row_spec,
    out_shape=jax.ShapeDtypeStruct((N, D), jnp.float32),
)


def kernel(feat, edge_index, W0_self, W0_neigh, b0, ln_g, ln_b,
           W1_self, W1_neigh, b1):
    src = edge_index[0].astype(jnp.int32)
    dst = edge_index[1].astype(jnp.int32)
    zrows = jnp.zeros((N, D), jnp.float32)
    ones = jnp.ones((K, D), jnp.float32)

    deg = _deg(dst, zrows, ones)
    agg0 = _agg(feat, src, dst, zrows)
    h1 = _tc_layer0(feat, agg0, deg, W0_self, W0_neigh,
                    b0.reshape(1, D), ln_g.reshape(1, D), ln_b.reshape(1, D))
    agg1 = _agg(h1, src, dst, zrows)
    out = _tc_layer1(h1, agg1, deg, W1_self, W1_neigh, b1.reshape(1, D))
    return out


# speedup vs baseline: 6.4485x; 1.0928x over previous
import functools

import jax
import jax.numpy as jnp
from jax import lax
from jax.experimental import pallas as pl
from jax.experimental.pallas import tpu as pltpu
from jax.experimental.pallas import tpu_sc as plsc

N = 10000
E = 320000
D = 128

NC = 2
NS = 16
NW = NC * NS
EPW = E // NW
K = 80
CH = EPW // K
KA = 40
CHA = EPW // KA
RING = 5
TRB = 624
LAST0 = TRB * (NS - 1)
LASTR = N - LAST0


def _make_agg_kernel():
    mesh = plsc.VectorSubcoreMesh(core_axis_name="c", subcore_axis_name="s")
    scratch = (
        [pltpu.VMEM((RING, KA), jnp.int32)] +
        [pltpu.VMEM((RING, KA), jnp.int32)] +
        [pltpu.VMEM((RING, KA, D), jnp.float32)] +
        [pltpu.VMEM_SHARED((N, D), jnp.float32)] +
        [pltpu.SemaphoreType.DMA] * (3 * RING)
    )

    def body(h_hbm, srcr_hbm, dstr_hbm, zrows_hbm, agg_out,
             src_v, dst_v, rows_v, agg_sh, *sems):
        si = sems[0:RING]
        sg = sems[RING:2 * RING]
        ss = sems[2 * RING:3 * RING]
        c = lax.axis_index("c")
        s = lax.axis_index("s")
        wid = s * NC + c
        row0 = pl.multiple_of(s * TRB, 8)

        @pl.when(s < NS - 1)
        def _():
            pltpu.sync_copy(zrows_hbm.at[pl.ds(row0, TRB)],
                            agg_sh.at[pl.ds(row0, TRB)])

        @pl.when(s == NS - 1)
        def _():
            pltpu.sync_copy(zrows_hbm.at[pl.ds(LAST0, LASTR)],
                            agg_sh.at[pl.ds(LAST0, LASTR)])

        plsc.subcore_barrier()

        base = wid * EPW

        def fire_idx(b, chunk):
            off = pl.multiple_of(base + chunk * KA, 8)
            pltpu.async_copy(srcr_hbm.at[pl.ds(off, KA)], src_v.at[b], si[b])
            pltpu.async_copy(dstr_hbm.at[pl.ds(off, KA)], dst_v.at[b], si[b])

        def wait_idx(b):
            pltpu.make_async_copy(srcr_hbm.at[pl.ds(0, KA)], src_v.at[b],
                                  si[b]).wait()
            pltpu.make_async_copy(dstr_hbm.at[pl.ds(0, KA)], dst_v.at[b],
                                  si[b]).wait()

        def fire_gather(b):
            pltpu.async_copy(h_hbm.at[src_v.at[b]], rows_v.at[b], sg[b])

        def wait_gather(b):
            pltpu.make_async_copy(h_hbm.at[src_v.at[b]], rows_v.at[b],
                                  sg[b]).wait()

        def fire_scatter(b):
            pltpu.async_copy(rows_v.at[b], agg_sh.at[dst_v.at[b]], ss[b],
                             add=True)

        def wait_scatter(b):
            pltpu.make_async_copy(rows_v.at[b], agg_sh.at[dst_v.at[b]],
                                  ss[b]).wait()

        def step(i, carry):
            for u in range(RING):
                t = RING * i + u
                @pl.when(jnp.logical_and(t >= 3, t < CHA + 3))
                def _(u=u):
                    wait_scatter((u - 3) % RING)

                @pl.when(jnp.logical_and(t >= 2, t < CHA + 2))
                def _(u=u):
                    wait_gather((u - 2) % RING)
                    fire_scatter((u - 2) % RING)

                @pl.when(jnp.logical_and(t >= 1, t < CHA + 1))
                def _(u=u):
                    wait_idx((u - 1) % RING)
                    fire_gather((u - 1) % RING)

                @pl.when(t < CHA)
                def _(u=u, t=t):
                    fire_idx(u, t)
            return carry

        lax.fori_loop(0, (CHA + 3 + RING - 1) // RING + 1, step, 0)
        plsc.subcore_barrier()

        @pl.when(s < NS - 1)
        def _():
            pltpu.sync_copy(agg_sh.at[pl.ds(row0, TRB)],
                            agg_out.at[c, pl.ds(row0, TRB)])

        @pl.when(s == NS - 1)
        def _():
            pltpu.sync_copy(agg_sh.at[pl.ds(LAST0, LASTR)],
                            agg_out.at[c, pl.ds(LAST0, LASTR)])

    return pl.kernel(body,
                     out_type=jax.ShapeDtypeStruct((NC, N, D), jnp.float32),
                     mesh=mesh, scratch_types=scratch)


def _make_deg_kernel():
    mesh = plsc.VectorSubcoreMesh(core_axis_name="c", subcore_axis_name="s")
    scratch = [
        pltpu.VMEM((2, K), jnp.int32),
        pltpu.VMEM((K, D), jnp.float32),
        pltpu.VMEM_SHARED((N, D), jnp.float32),
        pltpu.SemaphoreType.DMA,
        pltpu.SemaphoreType.DMA,
    ]

    def body(dstr_hbm, zrows_hbm, ones_hbm, deg_out, dst_v, ones_v, deg_sh,
             ss0, ss1):
        c = lax.axis_index("c")
        s = lax.axis_index("s")
        wid = s * NC + c
        row0 = pl.multiple_of(s * TRB, 8)

        @pl.when(s < NS - 1)
        def _():
            pltpu.sync_copy(zrows_hbm.at[pl.ds(row0, TRB)],
                            deg_sh.at[pl.ds(row0, TRB)])

        @pl.when(s == NS - 1)
        def _():
            pltpu.sync_copy(zrows_hbm.at[pl.ds(LAST0, LASTR)],
                            deg_sh.at[pl.ds(LAST0, LASTR)])

        pltpu.sync_copy(ones_hbm, ones_v)
        plsc.subcore_barrier()

        base = wid * EPW
        ss = (ss0, ss1)

        def load_idx(b, chunk):
            off = pl.multiple_of(base + chunk * K, 8)
            pltpu.sync_copy(dstr_hbm.at[pl.ds(off, K)], dst_v.at[b])

        def fire_scatter(b):
            pltpu.async_copy(ones_v, deg_sh.at[dst_v.at[b]], ss[b], add=True)

        def wait_scatter(b):
            pltpu.make_async_copy(ones_v, deg_sh.at[dst_v.at[b]], ss[b]).wait()

        load_idx(0, 0)

        def step(g2, carry):
            g0 = 2 * g2
            fire_scatter(0)

            @pl.when(g2 > 0)
            def _():
                wait_scatter(1)
            load_idx(1, g0 + 1)
            fire_scatter(1)
            wait_scatter(0)
            load_idx(0, g0 + 2)
            return carry

        lax.fori_loop(0, (CH - 1) // 2, step, 0)
        fire_scatter(0)
        wait_scatter(1)
        wait_scatter(0)
        plsc.subcore_barrier()

        @pl.when(s < NS - 1)
        def _():
            pltpu.sync_copy(deg_sh.at[pl.ds(row0, TRB)],
                            deg_out.at[c, pl.ds(row0, TRB)])

        @pl.when(s == NS - 1)
        def _():
            pltpu.sync_copy(deg_sh.at[pl.ds(LAST0, LASTR)],
                            deg_out.at[c, pl.ds(LAST0, LASTR)])

    return pl.kernel(body,
                     out_type=jax.ShapeDtypeStruct((NC, N, D), jnp.float32),
                     mesh=mesh, scratch_types=scratch)


_agg = _make_agg_kernel()
_deg = _make_deg_kernel()


R = 1000


def _tc_layer0_body(feat_ref, agg_ref, deg_ref, ws_ref, wn_ref, b_ref,
                    g_ref, beta_ref, out_ref):
    aggp = agg_ref[...]
    agg = aggp[0] + aggp[1]
    degp = deg_ref[...]
    deg = (degp[0] + degp[1])[:, 0:1]
    inv = 1.0 / jnp.maximum(deg, 1.0)
    mean = agg * inv
    h = (jnp.dot(feat_ref[...], ws_ref[...], preferred_element_type=jnp.float32)
         + jnp.dot(mean, wn_ref[...], preferred_element_type=jnp.float32)
         + b_ref[...])
    mu = jnp.mean(h, axis=-1, keepdims=True)
    var = jnp.mean((h - mu) ** 2, axis=-1, keepdims=True)
    hn = (h - mu) * lax.rsqrt(var + 1e-5) * g_ref[...] + beta_ref[...]
    out_ref[...] = jnp.maximum(hn, 0.0)


def _tc_layer1_body(h_ref, agg_ref, deg_ref, ws_ref, wn_ref, b_ref, out_ref):
    aggp = agg_ref[...]
    agg = aggp[0] + aggp[1]
    degp = deg_ref[...]
    deg = (degp[0] + degp[1])[:, 0:1]
    inv = 1.0 / jnp.maximum(deg, 1.0)
    mean = agg * inv
    out_ref[...] = (
        jnp.dot(h_ref[...], ws_ref[...], preferred_element_type=jnp.float32)
        + jnp.dot(mean, wn_ref[...], preferred_element_type=jnp.float32)
        + b_ref[...])


def _full(shape):
    return pl.BlockSpec(shape, lambda i: (0,) * len(shape))


_row_spec = pl.BlockSpec((R, D), lambda i: (i, 0))
_agg_spec = pl.BlockSpec((NC, R, D), lambda i: (0, i, 0))
_deg_spec = pl.BlockSpec((NC, R, D), lambda i: (0, i, 0))

_tc_layer0 = pl.pallas_call(
    _tc_layer0_body,
    grid=(N // R,),
    in_specs=[_row_spec, _agg_spec, _deg_spec, _full((D, D)), _full((D, D)),
              _full((1, D)), _full((1, D)), _full((1, D))],
    out_specs=_row_spec,
    out_shape=jax.ShapeDtypeStruct((N, D), jnp.float32),
)

_tc_layer1 = pl.pallas_call(
    _tc_layer1_body,
    grid=(N // R,),
    in_specs=[_row_spec, _agg_spec, _deg_spec, _full((D, D)), _full((D, D)),
              _full((1, D))],
    out_specs=_row_spec,
    out_shape=jax.ShapeDtypeStruct((N, D), jnp.float32),
)


def kernel(feat, edge_index, W0_self, W0_neigh, b0, ln_g, ln_b,
           W1_self, W1_neigh, b1):
    src = edge_index[0].astype(jnp.int32)
    dst = edge_index[1].astype(jnp.int32)
    zrows = jnp.zeros((N, D), jnp.float32)
    ones = jnp.ones((K, D), jnp.float32)

    deg = _deg(dst, zrows, ones)
    agg0 = _agg(feat, src, dst, zrows)
    h1 = _tc_layer0(feat, agg0, deg, W0_self, W0_neigh,
                    b0.reshape(1, D), ln_g.reshape(1, D), ln_b.reshape(1, D))
    agg1 = _agg(h1, src, dst, zrows)
    out = _tc_layer1(h1, agg1, deg, W1_self, W1_neigh, b1.reshape(1, D))
    return out


# speedup vs baseline: 9.3083x; 1.4435x over previous
import functools

import jax
import jax.numpy as jnp
from jax import lax
from jax.experimental import pallas as pl
from jax.experimental.pallas import tpu as pltpu
from jax.experimental.pallas import tpu_sc as plsc

N = 10000
E = 320000
D = 128

NC = 2
NS = 16
NW = NC * NS
EPW = E // NW
K = 80
CH = EPW // K
KA = 40
CHA = EPW // KA
RING = 5
TRB = 624
LAST0 = TRB * (NS - 1)
LASTR = N - LAST0


def _make_agg_kernel():
    mesh = plsc.VectorSubcoreMesh(core_axis_name="c", subcore_axis_name="s")
    scratch = (
        [pltpu.VMEM((RING, KA), jnp.int32)] +
        [pltpu.VMEM((RING, KA), jnp.int32)] +
        [pltpu.VMEM((RING, KA, D), jnp.float32)] +
        [pltpu.VMEM_SHARED((N, D), jnp.float32)] +
        [pltpu.SemaphoreType.DMA] * (3 * RING)
    )

    def body(h_hbm, srcr_hbm, dstr_hbm, zrows_hbm, agg_out,
             src_v, dst_v, rows_v, agg_sh, *sems):
        si = sems[0:RING]
        sg = sems[RING:2 * RING]
        ss = sems[2 * RING:3 * RING]
        c = lax.axis_index("c")
        s = lax.axis_index("s")
        wid = s * NC + c
        row0 = pl.multiple_of(s * TRB, 8)

        @pl.when(s < NS - 1)
        def _():
            pltpu.sync_copy(zrows_hbm.at[pl.ds(row0, TRB)],
                            agg_sh.at[pl.ds(row0, TRB)])

        @pl.when(s == NS - 1)
        def _():
            pltpu.sync_copy(zrows_hbm.at[pl.ds(LAST0, LASTR)],
                            agg_sh.at[pl.ds(LAST0, LASTR)])

        plsc.subcore_barrier()

        base = wid * EPW

        def fire_idx(b, chunk):
            off = pl.multiple_of(base + chunk * KA, 8)
            pltpu.async_copy(srcr_hbm.at[pl.ds(off, KA)], src_v.at[b], si[b])
            pltpu.async_copy(dstr_hbm.at[pl.ds(off, KA)], dst_v.at[b], si[b])

        def wait_idx(b):
            pltpu.make_async_copy(srcr_hbm.at[pl.ds(0, KA)], src_v.at[b],
                                  si[b]).wait()
            pltpu.make_async_copy(dstr_hbm.at[pl.ds(0, KA)], dst_v.at[b],
                                  si[b]).wait()

        def fire_gather(b):
            pltpu.async_copy(h_hbm.at[src_v.at[b]], rows_v.at[b], sg[b])

        def wait_gather(b):
            pltpu.make_async_copy(h_hbm.at[src_v.at[b]], rows_v.at[b],
                                  sg[b]).wait()

        def fire_scatter(b):
            pltpu.async_copy(rows_v.at[b], agg_sh.at[dst_v.at[b]], ss[b],
                             add=True)

        def wait_scatter(b):
            pltpu.make_async_copy(rows_v.at[b], agg_sh.at[dst_v.at[b]],
                                  ss[b]).wait()

        def step(i, carry):
            for u in range(RING):
                t = RING * i + u
                @pl.when(jnp.logical_and(t >= 4, t < CHA + 4))
                def _(u=u):
                    wait_scatter((u - 4) % RING)

                @pl.when(jnp.logical_and(t >= 3, t < CHA + 3))
                def _(u=u):
                    wait_gather((u - 3) % RING)
                    fire_scatter((u - 3) % RING)

                @pl.when(jnp.logical_and(t >= 1, t < CHA + 1))
                def _(u=u):
                    wait_idx((u - 1) % RING)
                    fire_gather((u - 1) % RING)

                @pl.when(t < CHA)
                def _(u=u, t=t):
                    fire_idx(u, t)
            return carry

        lax.fori_loop(0, (CHA + 4 + RING - 1) // RING + 1, step, 0)
        plsc.subcore_barrier()

        @pl.when(s < NS - 1)
        def _():
            pltpu.sync_copy(agg_sh.at[pl.ds(row0, TRB)],
                            agg_out.at[c, pl.ds(row0, TRB)])

        @pl.when(s == NS - 1)
        def _():
            pltpu.sync_copy(agg_sh.at[pl.ds(LAST0, LASTR)],
                            agg_out.at[c, pl.ds(LAST0, LASTR)])

    return pl.kernel(body,
                     out_type=jax.ShapeDtypeStruct((NC, N, D), jnp.float32),
                     mesh=mesh, scratch_types=scratch)


def _make_deg_kernel():
    mesh = plsc.VectorSubcoreMesh(core_axis_name="c", subcore_axis_name="s")
    scratch = [
        pltpu.VMEM((2, K), jnp.int32),
        pltpu.VMEM((K, D), jnp.float32),
        pltpu.VMEM_SHARED((N, D), jnp.float32),
        pltpu.SemaphoreType.DMA,
        pltpu.SemaphoreType.DMA,
    ]

    def body(dstr_hbm, zrows_hbm, ones_hbm, deg_out, dst_v, ones_v, deg_sh,
             ss0, ss1):
        c = lax.axis_index("c")
        s = lax.axis_index("s")
        wid = s * NC + c
        row0 = pl.multiple_of(s * TRB, 8)

        @pl.when(s < NS - 1)
        def _():
            pltpu.sync_copy(zrows_hbm.at[pl.ds(row0, TRB)],
                            deg_sh.at[pl.ds(row0, TRB)])

        @pl.when(s == NS - 1)
        def _():
            pltpu.sync_copy(zrows_hbm.at[pl.ds(LAST0, LASTR)],
                            deg_sh.at[pl.ds(LAST0, LASTR)])

        pltpu.sync_copy(ones_hbm, ones_v)
        plsc.subcore_barrier()

        base = wid * EPW
        ss = (ss0, ss1)

        def load_idx(b, chunk):
            off = pl.multiple_of(base + chunk * K, 8)
            pltpu.sync_copy(dstr_hbm.at[pl.ds(off, K)], dst_v.at[b])

        def fire_scatter(b):
            pltpu.async_copy(ones_v, deg_sh.at[dst_v.at[b]], ss[b], add=True)

        def wait_scatter(b):
            pltpu.make_async_copy(ones_v, deg_sh.at[dst_v.at[b]], ss[b]).wait()

        load_idx(0, 0)

        def step(g2, carry):
            g0 = 2 * g2
            fire_scatter(0)

            @pl.when(g2 > 0)
            def _():
                wait_scatter(1)
            load_idx(1, g0 + 1)
            fire_scatter(1)
            wait_scatter(0)
            load_idx(0, g0 + 2)
            return carry

        lax.fori_loop(0, (CH - 1) // 2, step, 0)
        fire_scatter(0)
        wait_scatter(1)
        wait_scatter(0)
        plsc.subcore_barrier()

        @pl.when(s < NS - 1)
        def _():
            pltpu.sync_copy(deg_sh.at[pl.ds(row0, TRB)],
                            deg_out.at[c, pl.ds(row0, TRB)])

        @pl.when(s == NS - 1)
        def _():
            pltpu.sync_copy(deg_sh.at[pl.ds(LAST0, LASTR)],
                            deg_out.at[c, pl.ds(LAST0, LASTR)])

    return pl.kernel(body,
                     out_type=jax.ShapeDtypeStruct((NC, N, D), jnp.float32),
                     mesh=mesh, scratch_types=scratch)


_agg = _make_agg_kernel()
_deg = _make_deg_kernel()


R = 1000


def _tc_layer0_body(feat_ref, agg_ref, deg_ref, ws_ref, wn_ref, b_ref,
                    g_ref, beta_ref, out_ref):
    aggp = agg_ref[...]
    agg = aggp[0] + aggp[1]
    degp = deg_ref[...]
    deg = (degp[0] + degp[1])[:, 0:1]
    inv = 1.0 / jnp.maximum(deg, 1.0)
    mean = agg * inv
    h = (jnp.dot(feat_ref[...], ws_ref[...], preferred_element_type=jnp.float32)
         + jnp.dot(mean, wn_ref[...], preferred_element_type=jnp.float32)
         + b_ref[...])
    mu = jnp.mean(h, axis=-1, keepdims=True)
    var = jnp.mean((h - mu) ** 2, axis=-1, keepdims=True)
    hn = (h - mu) * lax.rsqrt(var + 1e-5) * g_ref[...] + beta_ref[...]
    out_ref[...] = jnp.maximum(hn, 0.0)


def _tc_layer1_body(h_ref, agg_ref, deg_ref, ws_ref, wn_ref, b_ref, out_ref):
    aggp = agg_ref[...]
    agg = aggp[0] + aggp[1]
    degp = deg_ref[...]
    deg = (degp[0] + degp[1])[:, 0:1]
    inv = 1.0 / jnp.maximum(deg, 1.0)
    mean = agg * inv
    out_ref[...] = (
        jnp.dot(h_ref[...], ws_ref[...], preferred_element_type=jnp.float32)
        + jnp.dot(mean, wn_ref[...], preferred_element_type=jnp.float32)
        + b_ref[...])


def _full(shape):
    return pl.BlockSpec(shape, lambda i: (0,) * len(shape))


_row_spec = pl.BlockSpec((R, D), lambda i: (i, 0))
_agg_spec = pl.BlockSpec((NC, R, D), lambda i: (0, i, 0))
_deg_spec = pl.BlockSpec((NC, R, D), lambda i: (0, i, 0))

_tc_layer0 = pl.pallas_call(
    _tc_layer0_body,
    grid=(N // R,),
    in_specs=[_row_spec, _agg_spec, _deg_spec, _full((D, D)), _full((D, D)),
              _full((1, D)), _full((1, D)), _full((1, D))],
    out_specs=_row_spec,
    out_shape=jax.ShapeDtypeStruct((N, D), jnp.float32),
)

_tc_layer1 = pl.pallas_call(
    _tc_layer1_body,
    grid=(N // R,),
    in_specs=[_row_spec, _agg_spec, _deg_spec, _full((D, D)), _full((D, D)),
              _full((1, D))],
    out_specs=_row_spec,
    out_shape=jax.ShapeDtypeStruct((N, D), jnp.float32),
)


def kernel(feat, edge_index, W0_self, W0_neigh, b0, ln_g, ln_b,
           W1_self, W1_neigh, b1):
    src = edge_index[0].astype(jnp.int32)
    dst = edge_index[1].astype(jnp.int32)
    zrows = jnp.zeros((N, D), jnp.float32)
    ones = jnp.ones((K, D), jnp.float32)

    deg = _deg(dst, zrows, ones)
    agg0 = _agg(feat, src, dst, zrows)
    h1 = _tc_layer0(feat, agg0, deg, W0_self, W0_neigh,
                    b0.reshape(1, D), ln_g.reshape(1, D), ln_b.reshape(1, D))
    agg1 = _agg(h1, src, dst, zrows)
    out = _tc_layer1(h1, agg1, deg, W1_self, W1_neigh, b1.reshape(1, D))
    return out


# speedup vs baseline: 9.4994x; 1.0205x over previous
import functools

import jax
import jax.numpy as jnp
from jax import lax
from jax.experimental import pallas as pl
from jax.experimental.pallas import tpu as pltpu
from jax.experimental.pallas import tpu_sc as plsc

N = 10000
E = 320000
D = 128

NC = 2
NS = 16
NW = NC * NS
EPW = E // NW
K = 80
CH = EPW // K
KA = 40
CHA = EPW // KA
RING = 6
TRB = 624
LAST0 = TRB * (NS - 1)
LASTR = N - LAST0


def _make_agg_kernel():
    mesh = plsc.VectorSubcoreMesh(core_axis_name="c", subcore_axis_name="s")
    scratch = (
        [pltpu.VMEM((RING, KA), jnp.int32)] +
        [pltpu.VMEM((RING, KA), jnp.int32)] +
        [pltpu.VMEM((RING, KA, D), jnp.float32)] +
        [pltpu.VMEM_SHARED((N, D), jnp.float32)] +
        [pltpu.SemaphoreType.DMA] * (3 * RING)
    )

    def body(h_hbm, srcr_hbm, dstr_hbm, zrows_hbm, agg_out,
             src_v, dst_v, rows_v, agg_sh, *sems):
        si = sems[0:RING]
        sg = sems[RING:2 * RING]
        ss = sems[2 * RING:3 * RING]
        c = lax.axis_index("c")
        s = lax.axis_index("s")
        wid = s * NC + c
        row0 = pl.multiple_of(s * TRB, 8)

        @pl.when(s < NS - 1)
        def _():
            pltpu.sync_copy(zrows_hbm.at[pl.ds(row0, TRB)],
                            agg_sh.at[pl.ds(row0, TRB)])

        @pl.when(s == NS - 1)
        def _():
            pltpu.sync_copy(zrows_hbm.at[pl.ds(LAST0, LASTR)],
                            agg_sh.at[pl.ds(LAST0, LASTR)])

        plsc.subcore_barrier()

        base = wid * EPW

        def fire_idx(b, chunk):
            off = pl.multiple_of(base + chunk * KA, 8)
            pltpu.async_copy(srcr_hbm.at[pl.ds(off, KA)], src_v.at[b], si[b])
            pltpu.async_copy(dstr_hbm.at[pl.ds(off, KA)], dst_v.at[b], si[b])

        def wait_idx(b):
            pltpu.make_async_copy(srcr_hbm.at[pl.ds(0, KA)], src_v.at[b],
                                  si[b]).wait()
            pltpu.make_async_copy(dstr_hbm.at[pl.ds(0, KA)], dst_v.at[b],
                                  si[b]).wait()

        def fire_gather(b):
            pltpu.async_copy(h_hbm.at[src_v.at[b]], rows_v.at[b], sg[b])

        def wait_gather(b):
            pltpu.make_async_copy(h_hbm.at[src_v.at[b]], rows_v.at[b],
                                  sg[b]).wait()

        def fire_scatter(b):
            pltpu.async_copy(rows_v.at[b], agg_sh.at[dst_v.at[b]], ss[b],
                             add=True)

        def wait_scatter(b):
            pltpu.make_async_copy(rows_v.at[b], agg_sh.at[dst_v.at[b]],
                                  ss[b]).wait()

        def step(i, carry):
            for u in range(RING):
                t = RING * i + u
                @pl.when(jnp.logical_and(t >= 5, t < CHA + 5))
                def _(u=u):
                    wait_scatter((u - 5) % RING)

                @pl.when(jnp.logical_and(t >= 4, t < CHA + 4))
                def _(u=u):
                    wait_gather((u - 4) % RING)
                    fire_scatter((u - 4) % RING)

                @pl.when(jnp.logical_and(t >= 1, t < CHA + 1))
                def _(u=u):
                    wait_idx((u - 1) % RING)
                    fire_gather((u - 1) % RING)

                @pl.when(t < CHA)
                def _(u=u, t=t):
                    fire_idx(u, t)
            return carry

        lax.fori_loop(0, (CHA + 5 + RING - 1) // RING + 1, step, 0)
        plsc.subcore_barrier()

        @pl.when(s < NS - 1)
        def _():
            pltpu.sync_copy(agg_sh.at[pl.ds(row0, TRB)],
                            agg_out.at[c, pl.ds(row0, TRB)])

        @pl.when(s == NS - 1)
        def _():
            pltpu.sync_copy(agg_sh.at[pl.ds(LAST0, LASTR)],
                            agg_out.at[c, pl.ds(LAST0, LASTR)])

    return pl.kernel(body,
                     out_type=jax.ShapeDtypeStruct((NC, N, D), jnp.float32),
                     mesh=mesh, scratch_types=scratch)


def _make_deg_kernel():
    mesh = plsc.VectorSubcoreMesh(core_axis_name="c", subcore_axis_name="s")
    scratch = [
        pltpu.VMEM((2, K), jnp.int32),
        pltpu.VMEM((K, D), jnp.float32),
        pltpu.VMEM_SHARED((N, D), jnp.float32),
        pltpu.SemaphoreType.DMA,
        pltpu.SemaphoreType.DMA,
    ]

    def body(dstr_hbm, zrows_hbm, ones_hbm, deg_out, dst_v, ones_v, deg_sh,
             ss0, ss1):
        c = lax.axis_index("c")
        s = lax.axis_index("s")
        wid = s * NC + c
        row0 = pl.multiple_of(s * TRB, 8)

        @pl.when(s < NS - 1)
        def _():
            pltpu.sync_copy(zrows_hbm.at[pl.ds(row0, TRB)],
                            deg_sh.at[pl.ds(row0, TRB)])

        @pl.when(s == NS - 1)
        def _():
            pltpu.sync_copy(zrows_hbm.at[pl.ds(LAST0, LASTR)],
                            deg_sh.at[pl.ds(LAST0, LASTR)])

        pltpu.sync_copy(ones_hbm, ones_v)
        plsc.subcore_barrier()

        base = wid * EPW
        ss = (ss0, ss1)

        def load_idx(b, chunk):
            off = pl.multiple_of(base + chunk * K, 8)
            pltpu.sync_copy(dstr_hbm.at[pl.ds(off, K)], dst_v.at[b])

        def fire_scatter(b):
            pltpu.async_copy(ones_v, deg_sh.at[dst_v.at[b]], ss[b], add=True)

        def wait_scatter(b):
            pltpu.make_async_copy(ones_v, deg_sh.at[dst_v.at[b]], ss[b]).wait()

        load_idx(0, 0)

        def step(g2, carry):
            g0 = 2 * g2
            fire_scatter(0)

            @pl.when(g2 > 0)
            def _():
                wait_scatter(1)
            load_idx(1, g0 + 1)
            fire_scatter(1)
            wait_scatter(0)
            load_idx(0, g0 + 2)
            return carry

        lax.fori_loop(0, (CH - 1) // 2, step, 0)
        fire_scatter(0)
        wait_scatter(1)
        wait_scatter(0)
        plsc.subcore_barrier()

        @pl.when(s < NS - 1)
        def _():
            pltpu.sync_copy(deg_sh.at[pl.ds(row0, TRB)],
                            deg_out.at[c, pl.ds(row0, TRB)])

        @pl.when(s == NS - 1)
        def _():
            pltpu.sync_copy(deg_sh.at[pl.ds(LAST0, LASTR)],
                            deg_out.at[c, pl.ds(LAST0, LASTR)])

    return pl.kernel(body,
                     out_type=jax.ShapeDtypeStruct((NC, N, D), jnp.float32),
                     mesh=mesh, scratch_types=scratch)


_agg = _make_agg_kernel()
_deg = _make_deg_kernel()


R = 1000


def _tc_layer0_body(feat_ref, agg_ref, deg_ref, ws_ref, wn_ref, b_ref,
                    g_ref, beta_ref, out_ref):
    aggp = agg_ref[...]
    agg = aggp[0] + aggp[1]
    degp = deg_ref[...]
    deg = (degp[0] + degp[1])[:, 0:1]
    inv = 1.0 / jnp.maximum(deg, 1.0)
    mean = agg * inv
    h = (jnp.dot(feat_ref[...], ws_ref[...], preferred_element_type=jnp.float32)
         + jnp.dot(mean, wn_ref[...], preferred_element_type=jnp.float32)
         + b_ref[...])
    mu = jnp.mean(h, axis=-1, keepdims=True)
    var = jnp.mean((h - mu) ** 2, axis=-1, keepdims=True)
    hn = (h - mu) * lax.rsqrt(var + 1e-5) * g_ref[...] + beta_ref[...]
    out_ref[...] = jnp.maximum(hn, 0.0)


def _tc_layer1_body(h_ref, agg_ref, deg_ref, ws_ref, wn_ref, b_ref, out_ref):
    aggp = agg_ref[...]
    agg = aggp[0] + aggp[1]
    degp = deg_ref[...]
    deg = (degp[0] + degp[1])[:, 0:1]
    inv = 1.0 / jnp.maximum(deg, 1.0)
    mean = agg * inv
    out_ref[...] = (
        jnp.dot(h_ref[...], ws_ref[...], preferred_element_type=jnp.float32)
        + jnp.dot(mean, wn_ref[...], preferred_element_type=jnp.float32)
        + b_ref[...])


def _full(shape):
    return pl.BlockSpec(shape, lambda i: (0,) * len(shape))


_row_spec = pl.BlockSpec((R, D), lambda i: (i, 0))
_agg_spec = pl.BlockSpec((NC, R, D), lambda i: (0, i, 0))
_deg_spec = pl.BlockSpec((NC, R, D), lambda i: (0, i, 0))

_tc_layer0 = pl.pallas_call(
    _tc_layer0_body,
    grid=(N // R,),
    in_specs=[_row_spec, _agg_spec, _deg_spec, _full((D, D)), _full((D, D)),
              _full((1, D)), _full((1, D)), _full((1, D))],
    out_specs=_row_spec,
    out_shape=jax.ShapeDtypeStruct((N, D), jnp.float32),
)

_tc_layer1 = pl.pallas_call(
    _tc_layer1_body,
    grid=(N // R,),
    in_specs=[_row_spec, _agg_spec, _deg_spec, _full((D, D)), _full((D, D)),
              _full((1, D))],
    out_specs=_row_spec,
    out_shape=jax.ShapeDtypeStruct((N, D), jnp.float32),
)


def kernel(feat, edge_index, W0_self, W0_neigh, b0, ln_g, ln_b,
           W1_self, W1_neigh, b1):
    src = edge_index[0].astype(jnp.int32)
    dst = edge_index[1].astype(jnp.int32)
    zrows = jnp.zeros((N, D), jnp.float32)
    ones = jnp.ones((K, D), jnp.float32)

    deg = _deg(dst, zrows, ones)
    agg0 = _agg(feat, src, dst, zrows)
    h1 = _tc_layer0(feat, agg0, deg, W0_self, W0_neigh,
                    b0.reshape(1, D), ln_g.reshape(1, D), ln_b.reshape(1, D))
    agg1 = _agg(h1, src, dst, zrows)
    out = _tc_layer1(h1, agg1, deg, W1_self, W1_neigh, b1.reshape(1, D))
    return out
